# speedup vs baseline: 1.4840x; 1.4840x over previous
import functools

import jax
import jax.numpy as jnp
from jax.experimental import pallas as pl
from jax.experimental.pallas import tpu as pltpu

T = 2048
H = 1024
I = 512
IS = 1024
E = 16

BM = 256
NM = T // BM


def _routed_body(rw_ref, x_ref, wg_ref, wu_ref, wd_ref, out_ref, acc_ref):
    e = pl.program_id(0)
    m = pl.program_id(1)

    x_blk = x_ref[pl.ds(m * BM, BM), :]
    logits = jax.lax.dot_general(
        x_blk, rw_ref[...], (((1,), (1,)), ((), ())),
        preferred_element_type=jnp.float32)
    mx = jnp.max(logits, axis=1, keepdims=True)
    col = jax.lax.broadcasted_iota(jnp.int32, logits.shape, 1)
    idx = jnp.min(jnp.where(logits == mx, col, E), axis=1, keepdims=True)
    wt = jnp.where(idx == e, jax.nn.sigmoid(mx), 0.0)

    xs = x_blk * wt
    g = jax.lax.dot_general(xs, wg_ref[0], (((1,), (1,)), ((), ())),
                            preferred_element_type=jnp.float32)
    u = jax.lax.dot_general(xs, wu_ref[0], (((1,), (1,)), ((), ())),
                            preferred_element_type=jnp.float32)
    p = (g * jax.nn.sigmoid(g)) * u
    contrib = jax.lax.dot_general(p, wd_ref[0], (((1,), (1,)), ((), ())),
                                  preferred_element_type=jnp.float32)

    @pl.when(e == 0)
    def _init():
        acc_ref[pl.ds(m * BM, BM), :] = contrib

    @pl.when(e != 0)
    def _acc():
        acc_ref[pl.ds(m * BM, BM), :] = acc_ref[pl.ds(m * BM, BM), :] + contrib

    out_ref[...] = acc_ref[pl.ds(m * BM, BM), :]


def _shared_body(x_ref, sg_ref, su_ref, sd_ref, routed_ref, out_ref):
    x_blk = x_ref[...]
    g = jax.lax.dot_general(x_blk, sg_ref[...], (((1,), (1,)), ((), ())),
                            preferred_element_type=jnp.float32)
    u = jax.lax.dot_general(x_blk, su_ref[...], (((1,), (1,)), ((), ())),
                            preferred_element_type=jnp.float32)
    p = (g * jax.nn.sigmoid(g)) * u
    out_ref[...] = routed_ref[...] + jax.lax.dot_general(
        p, sd_ref[...], (((1,), (1,)), ((), ())),
        preferred_element_type=jnp.float32)


def kernel(hidden_states, router_weight, w_gate, w_up, w_down,
           shared_gate, shared_up, shared_down):
    routed = pl.pallas_call(
        _routed_body,
        grid=(E, NM),
        in_specs=[
            pl.BlockSpec((E, H), lambda e, m: (0, 0)),
            pl.BlockSpec((T, H), lambda e, m: (0, 0)),
            pl.BlockSpec((1, I, H), lambda e, m: (e, 0, 0)),
            pl.BlockSpec((1, I, H), lambda e, m: (e, 0, 0)),
            pl.BlockSpec((1, H, I), lambda e, m: (e, 0, 0)),
        ],
        out_specs=pl.BlockSpec((BM, H), lambda e, m: (m, 0)),
        out_shape=jax.ShapeDtypeStruct((T, H), jnp.float32),
        scratch_shapes=[pltpu.VMEM((T, H), jnp.float32)],
        compiler_params=pltpu.CompilerParams(
            dimension_semantics=("arbitrary", "arbitrary")),
    )(router_weight, hidden_states, w_gate, w_up, w_down)

    out = pl.pallas_call(
        _shared_body,
        grid=(NM,),
        in_specs=[
            pl.BlockSpec((BM, H), lambda m: (m, 0)),
            pl.BlockSpec((IS, H), lambda m: (0, 0)),
            pl.BlockSpec((IS, H), lambda m: (0, 0)),
            pl.BlockSpec((H, IS), lambda m: (0, 0)),
            pl.BlockSpec((BM, H), lambda m: (m, 0)),
        ],
        out_specs=pl.BlockSpec((BM, H), lambda m: (m, 0)),
        out_shape=jax.ShapeDtypeStruct((T, H), jnp.float32),
        compiler_params=pltpu.CompilerParams(
            dimension_semantics=("arbitrary",)),
    )(hidden_states, shared_gate, shared_up, shared_down, routed)
    return out


# speedup vs baseline: 1.8661x; 1.2574x over previous
import functools

import jax
import jax.numpy as jnp
from jax import lax
from jax.experimental import pallas as pl
from jax.experimental.pallas import tpu as pltpu
from jax.experimental.pallas import tpu_sc as plsc

T = 2048
H = 1024
I = 512
IS = 1024
E = 16

BM = 256
NM = T // BM
BM2 = 128
NM2 = T // BM2
J2 = NM2 + 1
NW = 32
CHUNK = T // NW


def _prep_body(rw_ref, x_ref, xs_ref, eid_ref, rank_ref, oinc_ref, oexc_ref,
               rc_ref):
    m = pl.program_id(0)
    x_blk = x_ref[...]
    logits = lax.dot_general(x_blk, rw_ref[...], (((1,), (1,)), ((), ())),
                             preferred_element_type=jnp.float32)
    mx = jnp.max(logits, axis=1, keepdims=True)
    col = lax.broadcasted_iota(jnp.int32, (BM, E), 1)
    idx = jnp.min(jnp.where(logits == mx, col, E), axis=1, keepdims=True)
    wt = jax.nn.sigmoid(mx)
    xs_ref[...] = x_blk * wt

    onehot = jnp.where(col == idx, 1.0, 0.0)

    @pl.when(m == 0)
    def _():
        rc_ref[...] = jnp.zeros((1, E), jnp.float32)

    ri = lax.broadcasted_iota(jnp.int32, (BM, BM), 0)
    ci = lax.broadcasted_iota(jnp.int32, (BM, BM), 1)
    lt = jnp.where(ci < ri, 1.0, 0.0)
    ranks_blk = lax.dot_general(lt, onehot, (((1,), (0,)), ((), ())),
                                preferred_element_type=jnp.float32)
    rank_rows = jnp.sum(onehot * (ranks_blk + rc_ref[...]), axis=1,
                        keepdims=True)

    eye = jnp.where(ci == ri, 1.0, 0.0)
    rank_t = lax.dot_general(rank_rows, eye, (((0,), (0,)), ((), ())),
                             preferred_element_type=jnp.float32)
    eid_t = lax.dot_general(idx.astype(jnp.float32), eye,
                            (((0,), (0,)), ((), ())),
                            preferred_element_type=jnp.float32)
    rank_ref[pl.ds(m, 1), :] = rank_t.astype(jnp.int32)
    eid_ref[pl.ds(m, 1), :] = eid_t.astype(jnp.int32)

    rc_new = rc_ref[...] + jnp.sum(onehot, axis=0, keepdims=True)
    rc_ref[...] = rc_new

    @pl.when(m == NM - 1)
    def _():
        ri16 = lax.broadcasted_iota(jnp.int32, (E, E), 0)
        ci16 = lax.broadcasted_iota(jnp.int32, (E, E), 1)
        ut = jnp.where(ri16 <= ci16, 1.0, 0.0)
        oinc = lax.dot_general(rc_new, ut, (((1,), (0,)), ((), ())),
                               preferred_element_type=jnp.float32)
        oinc_ref[...] = oinc.astype(jnp.int32)
        oexc_ref[...] = (oinc - rc_new).astype(jnp.int32)


def _route_prep(x, rw):
    return pl.pallas_call(
        _prep_body,
        grid=(NM,),
        in_specs=[
            pl.BlockSpec((E, H), lambda m: (0, 0)),
            pl.BlockSpec((BM, H), lambda m: (m, 0)),
        ],
        out_specs=[
            pl.BlockSpec((BM, H), lambda m: (m, 0)),
            pl.BlockSpec((NM, BM), lambda m: (0, 0)),
            pl.BlockSpec((NM, BM), lambda m: (0, 0)),
            pl.BlockSpec((1, E), lambda m: (0, 0)),
            pl.BlockSpec((1, E), lambda m: (0, 0)),
        ],
        out_shape=[
            jax.ShapeDtypeStruct((T, H), jnp.float32),
            jax.ShapeDtypeStruct((NM, BM), jnp.int32),
            jax.ShapeDtypeStruct((NM, BM), jnp.int32),
            jax.ShapeDtypeStruct((1, E), jnp.int32),
            jax.ShapeDtypeStruct((1, E), jnp.int32),
        ],
        scratch_shapes=[pltpu.VMEM((1, E), jnp.float32)],
        compiler_params=pltpu.CompilerParams(
            dimension_semantics=("arbitrary",)),
    )(rw, x)


def _pos_body(oexc_ref, eid_ref, rank_ref, pos_ref):
    p = rank_ref[...]
    ids = eid_ref[...]
    for e in range(E):
        p = jnp.where(ids == e, p + oexc_ref[0, e], p)
    pos_ref[...] = p


def _pos_calc(eid2d, rank2d, oexc2d):
    return pl.pallas_call(
        _pos_body,
        in_specs=[
            pl.BlockSpec(memory_space=pltpu.SMEM),
            pl.BlockSpec(),
            pl.BlockSpec(),
        ],
        out_specs=pl.BlockSpec(),
        out_shape=jax.ShapeDtypeStruct((NM, BM), jnp.int32),
    )(oexc2d, eid2d, rank2d)


def _sc_scatter_body(pos_hbm, xs_hbm, xsort_hbm, posv, rows, sem):
    wid = lax.axis_index("s") * 2 + lax.axis_index("c")
    base = wid * CHUNK
    pltpu.sync_copy(pos_hbm.at[pl.ds(base, CHUNK)], posv)
    pltpu.sync_copy(xs_hbm.at[pl.ds(base, CHUNK)], rows)
    pltpu.async_copy(rows, xsort_hbm.at[posv], sem).wait()


def _sc_scatter(pos, xs):
    mesh = plsc.VectorSubcoreMesh(core_axis_name="c", subcore_axis_name="s")
    return pl.kernel(
        _sc_scatter_body,
        out_type=jax.ShapeDtypeStruct((T, H), jnp.float32),
        mesh=mesh,
        scratch_types=[
            pltpu.VMEM((CHUNK,), jnp.int32),
            pltpu.VMEM((CHUNK, H), jnp.float32),
            pltpu.SemaphoreType.DMA,
        ],
    )(pos, xs)


def _grouped_body(oexc_sref, oinc_sref, xs_ref, wg_ref, wu_ref, wd_ref,
                  out_ref, acc_ref):
    e = pl.program_id(0)
    j = pl.program_id(1)
    oe = oexc_sref[e]
    oi = oinc_sref[e]
    mraw = oe // BM2 + j
    mblk = jnp.clip(mraw, 0, NM2 - 1)
    active = (mraw * BM2 < oi) & (mraw < NM2) & (oi > oe)

    @pl.when((e == 0) & (j == 0))
    def _():
        acc_ref[...] = jnp.zeros((T, H), jnp.float32)

    @pl.when(active)
    def _():
        xb = xs_ref[...]
        g = lax.dot_general(xb, wg_ref[0], (((1,), (1,)), ((), ())),
                            preferred_element_type=jnp.float32)
        u = lax.dot_general(xb, wu_ref[0], (((1,), (1,)), ((), ())),
                            preferred_element_type=jnp.float32)
        rowg = mblk * BM2 + lax.broadcasted_iota(jnp.int32, (BM2, 1), 0)
        mask = jnp.where((rowg >= oe) & (rowg < oi), 1.0, 0.0)
        p = (g * jax.nn.sigmoid(g)) * u * mask
        contrib = lax.dot_general(p, wd_ref[0], (((1,), (1,)), ((), ())),
                                  preferred_element_type=jnp.float32)
        sl = pl.ds(mblk * BM2, BM2)
        acc_ref[sl, :] = acc_ref[sl, :] + contrib

    @pl.when(e == E - 1)
    def _():
        fb = jnp.minimum(j, NM2 - 1)
        out_ref[...] = acc_ref[pl.ds(fb * BM2, BM2), :]


def _grouped(oexc, oinc, xs_sorted, w_gate, w_up, w_down):
    grid_spec = pltpu.PrefetchScalarGridSpec(
        num_scalar_prefetch=2,
        grid=(E, J2),
        in_specs=[
            pl.BlockSpec((BM2, H),
                         lambda e, j, oexc, oinc:
                         (jnp.clip(oexc[e] // BM2 + j, 0, NM2 - 1), 0)),
            pl.BlockSpec((1, I, H), lambda e, j, oexc, oinc: (e, 0, 0)),
            pl.BlockSpec((1, I, H), lambda e, j, oexc, oinc: (e, 0, 0)),
            pl.BlockSpec((1, H, I), lambda e, j, oexc, oinc: (e, 0, 0)),
        ],
        out_specs=pl.BlockSpec(
            (BM2, H),
            lambda e, j, oexc, oinc:
            (jnp.where(e == E - 1, jnp.minimum(j, NM2 - 1), 0), 0)),
        scratch_shapes=[pltpu.VMEM((T, H), jnp.float32)],
    )
    return pl.pallas_call(
        _grouped_body,
        grid_spec=grid_spec,
        out_shape=jax.ShapeDtypeStruct((T, H), jnp.float32),
        compiler_params=pltpu.CompilerParams(
            dimension_semantics=("arbitrary", "arbitrary")),
    )(oexc, oinc, xs_sorted, w_gate, w_up, w_down)


def _shared_body(x_ref, sg_ref, su_ref, sd_ref, out_ref):
    x_blk = x_ref[...]
    g = lax.dot_general(x_blk, sg_ref[...], (((1,), (1,)), ((), ())),
                        preferred_element_type=jnp.float32)
    u = lax.dot_general(x_blk, su_ref[...], (((1,), (1,)), ((), ())),
                        preferred_element_type=jnp.float32)
    p = (g * jax.nn.sigmoid(g)) * u
    out_ref[...] = lax.dot_general(p, sd_ref[...], (((1,), (1,)), ((), ())),
                                   preferred_element_type=jnp.float32)


def _shared(x, sg, su, sd):
    return pl.pallas_call(
        _shared_body,
        grid=(NM,),
        in_specs=[
            pl.BlockSpec((BM, H), lambda m: (m, 0)),
            pl.BlockSpec((IS, H), lambda m: (0, 0)),
            pl.BlockSpec((IS, H), lambda m: (0, 0)),
            pl.BlockSpec((H, IS), lambda m: (0, 0)),
        ],
        out_specs=pl.BlockSpec((BM, H), lambda m: (m, 0)),
        out_shape=jax.ShapeDtypeStruct((T, H), jnp.float32),
        compiler_params=pltpu.CompilerParams(
            dimension_semantics=("arbitrary",)),
    )(x, sg, su, sd)


def _sc_gather_body(pos_hbm, rsort_hbm, routed_hbm, posv, rows, sem):
    wid = lax.axis_index("s") * 2 + lax.axis_index("c")
    base = wid * CHUNK
    pltpu.sync_copy(pos_hbm.at[pl.ds(base, CHUNK)], posv)
    pltpu.async_copy(rsort_hbm.at[posv], rows, sem).wait()
    pltpu.sync_copy(rows, routed_hbm.at[pl.ds(base, CHUNK)])


def _sc_gather(pos, rsort):
    mesh = plsc.VectorSubcoreMesh(core_axis_name="c", subcore_axis_name="s")
    return pl.kernel(
        _sc_gather_body,
        out_type=jax.ShapeDtypeStruct((T, H), jnp.float32),
        mesh=mesh,
        scratch_types=[
            pltpu.VMEM((CHUNK,), jnp.int32),
            pltpu.VMEM((CHUNK, H), jnp.float32),
            pltpu.SemaphoreType.DMA,
        ],
    )(pos, rsort)


def _add_body(a_ref, b_ref, o_ref):
    o_ref[...] = a_ref[...] + b_ref[...]


def _final_add(a, b):
    return pl.pallas_call(
        _add_body,
        grid=(NM,),
        in_specs=[
            pl.BlockSpec((BM, H), lambda m: (m, 0)),
            pl.BlockSpec((BM, H), lambda m: (m, 0)),
        ],
        out_specs=pl.BlockSpec((BM, H), lambda m: (m, 0)),
        out_shape=jax.ShapeDtypeStruct((T, H), jnp.float32),
    )(a, b)


def kernel(hidden_states, router_weight, w_gate, w_up, w_down,
           shared_gate, shared_up, shared_down):
    xs, eid2d, rank2d, oinc2d, oexc2d = _route_prep(hidden_states,
                                                    router_weight)
    oinc = oinc2d.reshape(E)
    oexc = oexc2d.reshape(E)
    pos = _pos_calc(eid2d, rank2d, oexc2d).reshape(T)

    xsort = _sc_scatter(pos, xs)
    shared_out = _shared(hidden_states, shared_gate, shared_up, shared_down)
    rsort = _grouped(oexc, oinc, xsort, w_gate, w_up, w_down)
    routed = _sc_gather(pos, rsort)
    return _final_add(shared_out, routed)
